# baseline (speedup 1.0000x reference)
import functools

import jax
import jax.numpy as jnp
from jax import lax
from jax.experimental import pallas as pl
from jax.experimental.pallas import tpu as pltpu
from jax.experimental.pallas import tpu_sc as plsc

N = 10000
E = 160000
D = 256
DE = 16
DG = 128
H = 512
DOUT = 256

_BF = jnp.bfloat16


_PROJ_BN = 1000


def _proj_kernel(x_ref, w_ref, o_ref):
    o_ref[...] = jnp.dot(
        x_ref[...].astype(_BF), w_ref[...].astype(_BF),
        preferred_element_type=jnp.float32).astype(_BF)


def _project_nodes(x, w_sr):
    nb = N // _PROJ_BN
    return pl.pallas_call(
        _proj_kernel,
        grid=(2, nb),
        in_specs=[
            pl.BlockSpec((_PROJ_BN, D), lambda j, i: (i, 0)),
            pl.BlockSpec((D, H), lambda j, i: (0, j)),
        ],
        out_specs=pl.BlockSpec((_PROJ_BN, H), lambda j, i: (j * nb + i, 0)),
        out_shape=jax.ShapeDtypeStruct((2 * N, H), _BF),
    )(x, w_sr)


_NC = 2
_NS = 16
_NW = _NC * _NS
_B_TOTAL = 2 * E
_PER_W = _B_TOTAL // _NW
_CHUNK = 80
_NCHUNK = _PER_W // _CHUNK


@functools.lru_cache(maxsize=None)
def _make_sc_gather():
    @functools.partial(
        pl.kernel,
        mesh=plsc.VectorSubcoreMesh(core_axis_name="c", subcore_axis_name="s"),
        out_type=jax.ShapeDtypeStruct((_B_TOTAL, H // 2), jnp.int32),
        scratch_types=[
            pltpu.VMEM((_CHUNK,), jnp.int32),
            pltpu.VMEM((_CHUNK, H // 2), jnp.int32),
            pltpu.SemaphoreType.DMA,
        ],
    )
    def _sc_gather(t_hbm, i_hbm, o_hbm, idx_v, rows_v, sem):
        wid = lax.axis_index("s") * _NC + lax.axis_index("c")
        base = wid * _PER_W

        @pl.loop(0, _NCHUNK)
        def _(ci):
            off = base + ci * _CHUNK
            pltpu.sync_copy(i_hbm.at[pl.ds(off, _CHUNK)], idx_v)
            pltpu.async_copy(t_hbm.at[idx_v], rows_v, sem).wait()
            pltpu.sync_copy(rows_v, o_hbm.at[pl.ds(off, _CHUNK)])

    return _sc_gather


_FIN_BE = 2000
_FIN_NB = E // _FIN_BE


def _finalize_kernel(s_ref, r_ref, ea_ref, g_ref, w1e_ref, w1g_ref, b1_ref,
                     w2_ref, b2_ref, o_ref):
    msg = s_ref[...].astype(jnp.float32) + r_ref[...].astype(jnp.float32)
    eap = jnp.dot(ea_ref[...].astype(_BF), w1e_ref[...].astype(_BF),
                  preferred_element_type=jnp.float32)
    gp = jnp.dot(g_ref[...].astype(_BF), w1g_ref[...].astype(_BF),
                 preferred_element_type=jnp.float32)
    h = jnp.maximum(msg + eap + (gp + b1_ref[...]), 0.0)
    o_ref[...] = jnp.dot(h.astype(_BF), w2_ref[...].astype(_BF),
                         preferred_element_type=jnp.float32) + b2_ref[...]


def _finalize(gathered, ea, g, w1e, w1g, b1, w2, b2):
    return pl.pallas_call(
        _finalize_kernel,
        grid=(_FIN_NB,),
        in_specs=[
            pl.BlockSpec((_FIN_BE, H), lambda i: (i, 0)),
            pl.BlockSpec((_FIN_BE, H), lambda i: (i + _FIN_NB, 0)),
            pl.BlockSpec((_FIN_BE, DE), lambda i: (i, 0)),
            pl.BlockSpec((1, DG), lambda i: (0, 0)),
            pl.BlockSpec((DE, H), lambda i: (0, 0)),
            pl.BlockSpec((DG, H), lambda i: (0, 0)),
            pl.BlockSpec((1, H), lambda i: (0, 0)),
            pl.BlockSpec((H, DOUT), lambda i: (0, 0)),
            pl.BlockSpec((1, DOUT), lambda i: (0, 0)),
        ],
        out_specs=pl.BlockSpec((_FIN_BE, DOUT), lambda i: (i, 0)),
        out_shape=jax.ShapeDtypeStruct((E, DOUT), jnp.float32),
    )(gathered, gathered, ea, g, w1e, w1g, b1, w2, b2)


def kernel(node_attributes, edge_index, edge_attributes, global_attributes,
           W1, b1, W2, b2):
    w1e = W1[:DE]
    w1s = W1[DE:DE + D]
    w1r = W1[DE + D:DE + 2 * D]
    w1g = W1[DE + 2 * D:]
    w_sr = jnp.concatenate([w1s, w1r], axis=1)

    table = _project_nodes(node_attributes, w_sr)
    table_i32 = jax.lax.bitcast_convert_type(
        table.reshape(2 * N, H // 2, 2), jnp.int32)

    idx = jnp.concatenate([edge_index[0], edge_index[1] + N]).astype(jnp.int32)
    gathered_i32 = _make_sc_gather()(table_i32, idx)
    gathered = jax.lax.bitcast_convert_type(gathered_i32, _BF).reshape(
        2 * E, H)

    return _finalize(
        gathered, edge_attributes, global_attributes.reshape(1, DG),
        w1e, w1g, b1.reshape(1, H), W2, b2.reshape(1, DOUT))


# speedup vs baseline: 5.6512x; 5.6512x over previous
import functools

import jax
import jax.numpy as jnp
from jax import lax
from jax.experimental import pallas as pl
from jax.experimental.pallas import tpu as pltpu
from jax.experimental.pallas import tpu_sc as plsc

N = 10000
E = 160000
D = 256
DE = 16
DG = 128
H = 512
DOUT = 256

_BF = jnp.bfloat16


_PROJ_BN = 1000


def _proj_kernel(x_ref, w_ref, o_ref):
    acc = jnp.dot(
        x_ref[...].astype(_BF), w_ref[...].astype(_BF),
        preferred_element_type=jnp.float32)
    bits = jax.lax.bitcast_convert_type(
        acc.astype(_BF).astype(jnp.float32), jnp.int32) >> 16
    lo = bits[:, :H // 2] & jnp.int32(0xFFFF)
    hi = bits[:, H // 2:] << 16
    o_ref[...] = hi | lo


def _project_nodes(x, w_sr):
    nb = N // _PROJ_BN
    return pl.pallas_call(
        _proj_kernel,
        grid=(2, nb),
        in_specs=[
            pl.BlockSpec((_PROJ_BN, D), lambda j, i: (i, 0)),
            pl.BlockSpec((D, H), lambda j, i: (0, j)),
        ],
        out_specs=pl.BlockSpec((_PROJ_BN, H // 2), lambda j, i: (j * nb + i, 0)),
        out_shape=jax.ShapeDtypeStruct((2 * N, H // 2), jnp.int32),
    )(x, w_sr)


_NC = 2
_NS = 16
_NW = _NC * _NS
_B_TOTAL = 2 * E
_PER_W = _B_TOTAL // _NW
_CHUNK = 80
_NCHUNK = _PER_W // _CHUNK


@functools.lru_cache(maxsize=None)
def _make_sc_gather():
    @functools.partial(
        pl.kernel,
        mesh=plsc.VectorSubcoreMesh(core_axis_name="c", subcore_axis_name="s"),
        out_type=jax.ShapeDtypeStruct((_B_TOTAL, H // 2), jnp.int32),
        scratch_types=[
            pltpu.VMEM((_CHUNK,), jnp.int32),
            pltpu.VMEM((_CHUNK, H // 2), jnp.int32),
            pltpu.SemaphoreType.DMA,
        ],
    )
    def _sc_gather(t_hbm, i_hbm, o_hbm, idx_v, rows_v, sem):
        wid = lax.axis_index("s") * _NC + lax.axis_index("c")
        base = wid * _PER_W

        @pl.loop(0, _NCHUNK)
        def _(ci):
            off = base + ci * _CHUNK
            pltpu.sync_copy(i_hbm.at[pl.ds(off, _CHUNK)], idx_v)
            pltpu.async_copy(t_hbm.at[idx_v], rows_v, sem).wait()
            pltpu.sync_copy(rows_v, o_hbm.at[pl.ds(off, _CHUNK)])

    return _sc_gather


_FIN_BE = 2000
_FIN_NB = E // _FIN_BE


def _unpack_lo(v):
    return jax.lax.bitcast_convert_type(v << 16, jnp.float32)


def _unpack_hi(v):
    return jax.lax.bitcast_convert_type(v & jnp.int32(-65536), jnp.float32)


def _finalize_kernel(s_ref, r_ref, ea_ref, g_ref, w1e_ref, w1g_ref, b1_ref,
                     w2_ref, b2_ref, o_ref):
    s = s_ref[...]
    r = r_ref[...]
    eap = jnp.dot(ea_ref[...].astype(_BF), w1e_ref[...].astype(_BF),
                  preferred_element_type=jnp.float32)
    gp = jnp.dot(g_ref[...].astype(_BF), w1g_ref[...].astype(_BF),
                 preferred_element_type=jnp.float32)
    base = eap + (gp + b1_ref[...])
    hm = H // 2
    h_lo = jnp.maximum(_unpack_lo(s) + _unpack_lo(r) + base[:, :hm], 0.0)
    h_hi = jnp.maximum(_unpack_hi(s) + _unpack_hi(r) + base[:, hm:], 0.0)
    o_ref[...] = (
        jnp.dot(h_lo.astype(_BF), w2_ref[:hm, :].astype(_BF),
                preferred_element_type=jnp.float32)
        + jnp.dot(h_hi.astype(_BF), w2_ref[hm:, :].astype(_BF),
                  preferred_element_type=jnp.float32)
        + b2_ref[...])


def _finalize(gathered, ea, g, w1e, w1g, b1, w2, b2):
    return pl.pallas_call(
        _finalize_kernel,
        grid=(_FIN_NB,),
        in_specs=[
            pl.BlockSpec((_FIN_BE, H // 2), lambda i: (i, 0)),
            pl.BlockSpec((_FIN_BE, H // 2), lambda i: (i + _FIN_NB, 0)),
            pl.BlockSpec((_FIN_BE, DE), lambda i: (i, 0)),
            pl.BlockSpec((1, DG), lambda i: (0, 0)),
            pl.BlockSpec((DE, H), lambda i: (0, 0)),
            pl.BlockSpec((DG, H), lambda i: (0, 0)),
            pl.BlockSpec((1, H), lambda i: (0, 0)),
            pl.BlockSpec((H, DOUT), lambda i: (0, 0)),
            pl.BlockSpec((1, DOUT), lambda i: (0, 0)),
        ],
        out_specs=pl.BlockSpec((_FIN_BE, DOUT), lambda i: (i, 0)),
        out_shape=jax.ShapeDtypeStruct((E, DOUT), jnp.float32),
    )(gathered, gathered, ea, g, w1e, w1g, b1, w2, b2)


def kernel(node_attributes, edge_index, edge_attributes, global_attributes,
           W1, b1, W2, b2):
    w1e = W1[:DE]
    w1s = W1[DE:DE + D]
    w1r = W1[DE + D:DE + 2 * D]
    w1g = W1[DE + 2 * D:]
    w_sr = jnp.concatenate([w1s, w1r], axis=1)

    table = _project_nodes(node_attributes, w_sr)

    idx = jnp.concatenate([edge_index[0], edge_index[1] + N]).astype(jnp.int32)
    gathered = _make_sc_gather()(table, idx)

    return _finalize(
        gathered, edge_attributes, global_attributes.reshape(1, DG),
        w1e, w1g, b1.reshape(1, H), W2, b2.reshape(1, DOUT))


# speedup vs baseline: 7.1263x; 1.2610x over previous
import functools

import jax
import jax.numpy as jnp
from jax import lax
from jax.experimental import pallas as pl
from jax.experimental.pallas import tpu as pltpu
from jax.experimental.pallas import tpu_sc as plsc

N = 10000
E = 160000
D = 256
DE = 16
DG = 128
H = 512
DOUT = 256

_BF = jnp.bfloat16


_PROJ_BN = 1000


def _proj_kernel(x_ref, w_ref, o_ref):
    acc = jnp.dot(
        x_ref[...].astype(_BF), w_ref[...].astype(_BF),
        preferred_element_type=jnp.float32)
    bits = jax.lax.bitcast_convert_type(
        acc.astype(_BF).astype(jnp.float32), jnp.int32) >> 16
    lo = bits[:, :H // 2] & jnp.int32(0xFFFF)
    hi = bits[:, H // 2:] << 16
    o_ref[...] = hi | lo


def _project_nodes(x, w_sr):
    nb = N // _PROJ_BN
    return pl.pallas_call(
        _proj_kernel,
        grid=(2, nb),
        in_specs=[
            pl.BlockSpec((_PROJ_BN, D), lambda j, i: (i, 0)),
            pl.BlockSpec((D, H), lambda j, i: (0, j)),
        ],
        out_specs=pl.BlockSpec((_PROJ_BN, H // 2), lambda j, i: (j * nb + i, 0)),
        out_shape=jax.ShapeDtypeStruct((2 * N, H // 2), jnp.int32),
    )(x, w_sr)


_NC = 2
_NS = 16
_NW = _NC * _NS
_B_TOTAL = 2 * E
_PER_W = _B_TOTAL // _NW
_CHUNK = 80
_NCHUNK = _PER_W // _CHUNK


@functools.lru_cache(maxsize=None)
def _make_sc_gather():
    @functools.partial(
        pl.kernel,
        mesh=plsc.VectorSubcoreMesh(core_axis_name="c", subcore_axis_name="s"),
        out_type=jax.ShapeDtypeStruct((_B_TOTAL, H // 2), jnp.int32),
        scratch_types=[
            pltpu.VMEM((_PER_W,), jnp.int32),
            pltpu.VMEM((_CHUNK, H // 2), jnp.int32),
            pltpu.VMEM((_CHUNK, H // 2), jnp.int32),
            pltpu.SemaphoreType.DMA,
            pltpu.SemaphoreType.DMA,
        ],
    )
    def _sc_gather(t_hbm, i_hbm, o_hbm, idx_v, rows0, rows1, sem0, sem1):
        wid = lax.axis_index("s") * _NC + lax.axis_index("c")
        base = wid * _PER_W
        pltpu.sync_copy(i_hbm.at[pl.ds(base, _PER_W)], idx_v)

        def g_start(ci, rows, sem):
            pltpu.make_async_copy(
                t_hbm.at[idx_v.at[pl.ds(ci * _CHUNK, _CHUNK)]], rows, sem
            ).start()

        def g_wait(ci, rows, sem):
            pltpu.make_async_copy(
                t_hbm.at[idx_v.at[pl.ds(ci * _CHUNK, _CHUNK)]], rows, sem
            ).wait()

        def wb(ci, rows):
            pltpu.sync_copy(rows, o_hbm.at[pl.ds(base + ci * _CHUNK, _CHUNK)])

        g_start(0, rows0, sem0)

        @pl.loop(0, (_NCHUNK - 1) // 2)
        def _(cp):
            ci0 = 2 * cp
            ci1 = ci0 + 1
            g_start(ci1, rows1, sem1)
            g_wait(ci0, rows0, sem0)
            wb(ci0, rows0)
            g_start(ci0 + 2, rows0, sem0)
            g_wait(ci1, rows1, sem1)
            wb(ci1, rows1)

        g_wait(_NCHUNK - 1, rows0, sem0)
        wb(_NCHUNK - 1, rows0)

    return _sc_gather


_FIN_BE = 2000
_FIN_NB = E // _FIN_BE


def _unpack_lo(v):
    return jax.lax.bitcast_convert_type(v << 16, jnp.float32)


def _unpack_hi(v):
    return jax.lax.bitcast_convert_type(v & jnp.int32(-65536), jnp.float32)


def _finalize_kernel(s_ref, r_ref, ea_ref, g_ref, w1e_ref, w1g_ref, b1_ref,
                     w2_ref, b2_ref, o_ref):
    s = s_ref[...]
    r = r_ref[...]
    eap = jnp.dot(ea_ref[...].astype(_BF), w1e_ref[...].astype(_BF),
                  preferred_element_type=jnp.float32)
    gp = jnp.dot(g_ref[...].astype(_BF), w1g_ref[...].astype(_BF),
                 preferred_element_type=jnp.float32)
    base = eap + (gp + b1_ref[...])
    hm = H // 2
    h_lo = jnp.maximum(_unpack_lo(s) + _unpack_lo(r) + base[:, :hm], 0.0)
    h_hi = jnp.maximum(_unpack_hi(s) + _unpack_hi(r) + base[:, hm:], 0.0)
    o_ref[...] = (
        jnp.dot(h_lo.astype(_BF), w2_ref[:hm, :].astype(_BF),
                preferred_element_type=jnp.float32)
        + jnp.dot(h_hi.astype(_BF), w2_ref[hm:, :].astype(_BF),
                  preferred_element_type=jnp.float32)
        + b2_ref[...])


def _finalize(gathered, ea, g, w1e, w1g, b1, w2, b2):
    return pl.pallas_call(
        _finalize_kernel,
        grid=(_FIN_NB,),
        in_specs=[
            pl.BlockSpec((_FIN_BE, H // 2), lambda i: (i, 0)),
            pl.BlockSpec((_FIN_BE, H // 2), lambda i: (i + _FIN_NB, 0)),
            pl.BlockSpec((_FIN_BE, DE), lambda i: (i, 0)),
            pl.BlockSpec((1, DG), lambda i: (0, 0)),
            pl.BlockSpec((DE, H), lambda i: (0, 0)),
            pl.BlockSpec((DG, H), lambda i: (0, 0)),
            pl.BlockSpec((1, H), lambda i: (0, 0)),
            pl.BlockSpec((H, DOUT), lambda i: (0, 0)),
            pl.BlockSpec((1, DOUT), lambda i: (0, 0)),
        ],
        out_specs=pl.BlockSpec((_FIN_BE, DOUT), lambda i: (i, 0)),
        out_shape=jax.ShapeDtypeStruct((E, DOUT), jnp.float32),
    )(gathered, gathered, ea, g, w1e, w1g, b1, w2, b2)


def kernel(node_attributes, edge_index, edge_attributes, global_attributes,
           W1, b1, W2, b2):
    w1e = W1[:DE]
    w1s = W1[DE:DE + D]
    w1r = W1[DE + D:DE + 2 * D]
    w1g = W1[DE + 2 * D:]
    w_sr = jnp.concatenate([w1s, w1r], axis=1)

    table = _project_nodes(node_attributes, w_sr)

    idx = jnp.concatenate([edge_index[0], edge_index[1] + N]).astype(jnp.int32)
    gathered = _make_sc_gather()(table, idx)

    return _finalize(
        gathered, edge_attributes, global_attributes.reshape(1, DG),
        w1e, w1g, b1.reshape(1, H), W2, b2.reshape(1, DOUT))
